# baseline (speedup 1.0000x reference)
import functools

import jax
import jax.numpy as jnp
import numpy as np
from jax import lax
from jax.experimental import pallas as pl
from jax.experimental.pallas import tpu as pltpu
from jax.experimental.pallas import tpu_sc as plsc

_ANCHOR_BASES = np.array(
    [[-84.0, -40.0, 99.0, 55.0], [-176.0, -88.0, 191.0, 103.0],
     [-360.0, -184.0, 375.0, 199.0], [-56.0, -56.0, 71.0, 71.0],
     [-120.0, -120.0, 135.0, 135.0], [-248.0, -248.0, 263.0, 263.0],
     [-36.0, -80.0, 51.0, 95.0], [-80.0, -168.0, 95.0, 183.0],
     [-168.0, -344.0, 183.0, 359.0]], dtype=np.float32)
_STRIDE = 16
_PRE_NMS_TOP_N = 6000
_POST_NMS_TOP_N = 300
_NMS_THRESH = 0.7

_B = 4
_H = _W = 48
_A = 9
_HW = _H * _W
_HWB = _HW // 128
_N = _HW * _A
_AROWS = _A * _HWB
_ROWS = 168
_NPAD = _ROWS * 128
_G = _B * _NPAD
_CROWS = 48
_CPAD = _CROWS * 128
_CTOT = _B * _CPAD
_SLOT_TOT = _CTOT + 128
_NULL_IDX = _AROWS * 128
_NW = 32
_P1_BLKS = 32
_P2_ROWS = _CTOT // _NW


def _np_anchors():
    shift_x = np.arange(0, _W) * _STRIDE
    shift_y = np.arange(0, _H) * _STRIDE
    sx, sy = np.meshgrid(shift_x, shift_y)
    shifts = np.stack([sx.ravel(), sy.ravel(), sx.ravel(), sy.ravel()],
                      axis=1).astype(np.float32)
    anchors = _ANCHOR_BASES.reshape(1, _A, 4) + shifts.reshape(-1, 1, 4)
    return anchors.reshape(_N, 4)


_ANCHORS_NP = _np_anchors()


def _anchor_consts():
    ab = _ANCHOR_BASES
    w_a = ab[:, 2] - ab[:, 0] + 1.0
    h_a = ab[:, 3] - ab[:, 1] + 1.0
    cx_a = ab[:, 0] + 0.5 * w_a
    cy_a = ab[:, 1] + 0.5 * h_a
    hw = np.arange(_HW, dtype=np.float32)
    sx = (np.arange(_HW) % _W * _STRIDE).astype(np.float32)
    sy = (np.arange(_HW) // _W * _STRIDE).astype(np.float32)
    del hw
    out = np.zeros((4, _ROWS * 128), dtype=np.float32)
    for a in range(_A):
        sl = slice(a * _HW, (a + 1) * _HW)
        out[0, sl] = w_a[a]
        out[1, sl] = h_a[a]
        out[2, sl] = cx_a[a] + sx
        out[3, sl] = cy_a[a] + sy
    out[0, _N:] = 1.0
    out[1, _N:] = 1.0
    return out.reshape(4, _ROWS, 128)


_ANC4_NP = _anchor_consts()
_SRCG_NP = np.arange(_G, dtype=np.int32).reshape(_P1_BLKS, 21, 128)
_NULL_NP = np.full((_SLOT_TOT,), _NULL_IDX, dtype=np.int32)


def _stage_a_kernel(s_ref, d_ref, anc_ref, hm_ref, wm_ref,
                    x1o, y1o, x2o, y2o, slot_o):
    f32 = jnp.float32
    sv = s_ref[...].reshape(_B, _AROWS, 128)
    padm1 = jnp.full((_B, _ROWS - _AROWS, 128), -1.0, f32)
    padz = jnp.zeros((_B, _ROWS - _AROWS, 128), f32)
    scv = jnp.concatenate([sv, padm1], axis=1)
    dv = d_ref[...]

    def chan(k):
        rows = jnp.concatenate([dv[:, a * 4 + k] for a in range(_A)], axis=1)
        return jnp.concatenate([rows, padz], axis=1)

    dx = chan(0)
    dy = chan(1)
    dw = chan(2)
    dh = chan(3)

    WA = anc_ref[0][None]
    HA = anc_ref[1][None]
    CX = anc_ref[2][None]
    CY = anc_ref[3][None]
    hm = jnp.max(hm_ref[...], axis=(1, 2), keepdims=True)
    wm = jnp.max(wm_ref[...], axis=(1, 2), keepdims=True)

    pcx = dx * WA + CX
    pcy = dy * HA + CY
    pw = jnp.exp(dw) * WA
    ph = jnp.exp(dh) * HA
    x1 = jnp.minimum(jnp.maximum(pcx - 0.5 * pw, 0.0), wm)
    y1 = jnp.minimum(jnp.maximum(pcy - 0.5 * ph, 0.0), hm)
    x2 = jnp.minimum(jnp.maximum(pcx + 0.5 * pw, 0.0), wm)
    y2 = jnp.minimum(jnp.maximum(pcy + 0.5 * ph, 0.0), hm)

    lin = (lax.broadcasted_iota(jnp.int32, (_B, _ROWS, 128), 1) * 128
           + lax.broadcasted_iota(jnp.int32, (_B, _ROWS, 128), 2))

    bits = lax.bitcast_convert_type(scv, jnp.int32)
    K = _PRE_NMS_TOP_N

    def sel_body(t, pfx):
        cand = pfx | (jnp.int32(1) << (jnp.int32(30) - t))
        cnt = jnp.sum((bits >= cand).astype(jnp.int32), axis=(1, 2),
                      keepdims=True)
        return jnp.where(cnt >= K, cand, pfx)

    v = lax.fori_loop(0, 31, sel_body, jnp.zeros((_B, 1, 1), jnp.int32))

    gt = bits > v
    eq = bits == v
    cnt_gt = jnp.sum(gt.astype(jnp.int32), axis=(1, 2), keepdims=True)
    m = (K - cnt_gt).astype(f32)

    b0 = lax.broadcasted_iota(jnp.int32, (_HWB, _ROWS), 0)
    r1b = lax.broadcasted_iota(jnp.int32, (_HWB, _ROWS), 1)
    M = (((r1b % _HWB) == b0) & (r1b < _AROWS)).astype(f32)
    M2 = (((r1b % _HWB) < b0) & (r1b < _AROWS)).astype(f32)
    c0 = lax.broadcasted_iota(jnp.int32, (128, 128), 0)
    c1 = lax.broadcasted_iota(jnp.int32, (128, 128), 1)
    MU = (c0 < c1).astype(f32)
    r0 = lax.broadcasted_iota(jnp.int32, (_ROWS, _ROWS), 0)
    r1 = lax.broadcasted_iota(jnp.int32, (_ROWS, _ROWS), 1)
    Km = (((r0 % _HWB) == (r1 % _HWB)) & ((r1 // _HWB) < (r0 // _HWB))
          & (r1 < _AROWS) & (r0 < _AROWS)).astype(f32)
    zpad6 = jnp.zeros((_ROWS - _AROWS, 128), f32)

    def prefix_ref(maskf):
        lanepart = lax.dot(M, lax.dot(maskf, MU, preferred_element_type=f32),
                           preferred_element_type=f32)
        rowpart = jnp.sum(lax.dot(M2, maskf, preferred_element_type=f32),
                          axis=1, keepdims=True)
        p1 = rowpart + lanepart
        t1 = jnp.concatenate([p1] * _A + [zpad6], axis=0)
        t2 = lax.dot(Km, maskf, preferred_element_type=f32)
        return t1 + t2

    eqf = eq.astype(f32)
    pcs = [prefix_ref(eqf[i])[None] for i in range(_B)]
    pc = jnp.concatenate(pcs, axis=0)
    cand = gt | (eq & (pc < m))

    candf = cand.astype(f32)
    rks = [prefix_ref(candf[i])[None] for i in range(_B)]
    rank = jnp.concatenate(rks, axis=0).astype(jnp.int32)

    img_off = lax.broadcasted_iota(jnp.int32, (_B, 1, 1), 0) * _CPAD
    dummy = _CTOT + (lin % 128)
    slot = jnp.where(cand, img_off + rank, dummy)

    x1o[...] = x1
    y1o[...] = y1
    x2o[...] = x2
    y2o[...] = y2
    slot_o[...] = slot


def _sc_compact_body(slot_hbm, src_hbm, null_hbm,
                     x1_hbm, y1_hbm, x2_hbm, y2_hbm, sc_hbm,
                     ox1, oy1, ox2, oy2, osc,
                     idxsp, slot_v, src_v, null_v, idx_v, vals_v,
                     sem1, sem2):
    c = lax.axis_index("c")
    s = lax.axis_index("s")
    wid = c * 16 + s
    chunk = _SLOT_TOT // 16
    pltpu.sync_copy(null_hbm.at[pl.ds(s * chunk, chunk)], null_v)
    pltpu.sync_copy(null_v, idxsp.at[pl.ds(s * chunk, chunk)])
    for j0 in (0, 16):
        j = s + j0
        pltpu.sync_copy(slot_hbm.at[j], slot_v)
        pltpu.sync_copy(src_hbm.at[j], src_v)
        descs = []
        for k in range(21):
            descs.append(
                pltpu.async_copy(src_v.at[k], idxsp.at[slot_v.at[k]], sem1))
        for d in descs:
            d.wait()
    plsc.subcore_barrier()
    pltpu.sync_copy(idxsp.at[pl.ds(wid * _P2_ROWS, _P2_ROWS)], idx_v)
    srcs = (x1_hbm, y1_hbm, x2_hbm, y2_hbm, sc_hbm)
    outs = (ox1, oy1, ox2, oy2, osc)
    descs = []
    for a in range(5):
        for k in range(_P2_ROWS // 128):
            descs.append(
                pltpu.async_copy(srcs[a].at[idx_v.at[pl.ds(k * 128, 128)]],
                                 vals_v.at[a, pl.ds(k * 128, 128)], sem2))
    for d in descs:
        d.wait()
    for a in range(5):
        pltpu.sync_copy(vals_v.at[a],
                        outs[a].at[pl.ds(wid * _P2_ROWS, _P2_ROWS)])


@functools.cache
def _sc_compact_callable():
    return pl.kernel(
        _sc_compact_body,
        out_type=[jax.ShapeDtypeStruct((_CTOT,), jnp.float32)] * 5,
        mesh=plsc.VectorSubcoreMesh(core_axis_name="c", subcore_axis_name="s",
                                    num_cores=2, num_subcores=16),
        scratch_types=[
            pltpu.VMEM_SHARED((_SLOT_TOT,), jnp.int32),
            pltpu.VMEM((21, 128), jnp.int32),
            pltpu.VMEM((21, 128), jnp.int32),
            pltpu.VMEM((_SLOT_TOT // 16,), jnp.int32),
            pltpu.VMEM((_P2_ROWS,), jnp.int32),
            pltpu.VMEM((5, _P2_ROWS), jnp.float32),
            pltpu.SemaphoreType.DMA,
            pltpu.SemaphoreType.DMA,
        ],
        compiler_params=pltpu.CompilerParams(use_tc_tiling_on_sc=False),
    )


def _sc_compact(*args):
    return _sc_compact_callable()(*args)


def _stage_b_kernel(sc_ref, x1_ref, y1_ref, x2_ref, y2_ref,
                    so_ref, bo_ref):
    f32 = jnp.float32
    scv = sc_ref[...]
    x1 = x1_ref[...]
    y1 = y1_ref[...]
    x2 = x2_ref[...]
    y2 = y2_ref[...]
    areas = (x2 - x1 + 1.0) * (y2 - y1 + 1.0)
    lin = (lax.broadcasted_iota(jnp.int32, (_B, _CROWS, 128), 1) * 128
           + lax.broadcasted_iota(jnp.int32, (_B, _CROWS, 128), 2))

    i8 = lax.broadcasted_iota(jnp.int32, (8, 128), 0)
    i128 = lax.broadcasted_iota(jnp.int32, (8, 128), 1)
    img_id = lax.broadcasted_iota(jnp.int32, (_B, 1, 1), 0).astype(f32)
    BIG = jnp.int32(2 ** 30)

    def body(r, carry):
        ms, sa, xa, ya, x2a, y2a = carry
        mx = jnp.max(ms, axis=(1, 2), keepdims=True)
        validr = mx >= 0.0
        hit = ms == mx
        idx = jnp.min(jnp.where(hit, lin, BIG), axis=(1, 2), keepdims=True)
        sel = hit & (lin == idx)
        sm = sel.astype(f32)
        bx1 = jnp.sum(sm * x1, axis=(1, 2), keepdims=True)
        by1 = jnp.sum(sm * y1, axis=(1, 2), keepdims=True)
        bx2 = jnp.sum(sm * x2, axis=(1, 2), keepdims=True)
        by2 = jnp.sum(sm * y2, axis=(1, 2), keepdims=True)
        barea = (bx2 - bx1 + 1.0) * (by2 - by1 + 1.0)
        xx1 = jnp.maximum(bx1, x1)
        yy1 = jnp.maximum(by1, y1)
        xx2 = jnp.minimum(bx2, x2)
        yy2 = jnp.minimum(by2, y2)
        iw = jnp.maximum(0.0, xx2 - xx1 + 1.0)
        ih = jnp.maximum(0.0, yy2 - yy1 + 1.0)
        inter = iw * ih
        iou = inter / (barea + areas - inter)
        ms = jnp.where(iou > _NMS_THRESH, -1.0, ms)
        wmask = ((i8 == (r // 128)) & (i128 == (r % 128)))[None]
        sval = jnp.where(validr, mx, img_id)
        sa = jnp.where(wmask, sval, sa)
        xa = jnp.where(wmask, jnp.where(validr, bx1, 0.0), xa)
        ya = jnp.where(wmask, jnp.where(validr, by1, 0.0), ya)
        x2a = jnp.where(wmask, jnp.where(validr, bx2, 0.0), x2a)
        y2a = jnp.where(wmask, jnp.where(validr, by2, 0.0), y2a)
        return ms, sa, xa, ya, x2a, y2a

    z = jnp.zeros((_B, 8, 128), f32)
    _, sa, xa, ya, x2a, y2a = lax.fori_loop(
        0, _POST_NMS_TOP_N, body, (scv, z, z, z, z, z))
    so_ref[...] = sa
    bo_ref[:, 0] = xa
    bo_ref[:, 1] = ya
    bo_ref[:, 2] = x2a
    bo_ref[:, 3] = y2a


def kernel(scores, bbox_deltas, im_info):
    f32 = jnp.float32
    B = _B
    sv = scores[:, _A:, :, :].reshape(B, _A, _HWB, 128)
    dv = bbox_deltas.reshape(B, 4 * _A, _HWB, 128)
    anc4 = jnp.asarray(_ANC4_NP)
    hmb = jnp.broadcast_to((im_info[:, 0] - 1.0)[:, None, None], (B, 8, 128))
    wmb = jnp.broadcast_to((im_info[:, 1] - 1.0)[:, None, None], (B, 8, 128))

    x1d, y1d, x2d, y2d, slot = pl.pallas_call(
        _stage_a_kernel,
        out_shape=[
            jax.ShapeDtypeStruct((B, _ROWS, 128), f32),
            jax.ShapeDtypeStruct((B, _ROWS, 128), f32),
            jax.ShapeDtypeStruct((B, _ROWS, 128), f32),
            jax.ShapeDtypeStruct((B, _ROWS, 128), f32),
            jax.ShapeDtypeStruct((B, _ROWS, 128), jnp.int32),
        ],
    )(sv, dv, anc4, hmb, wmb)

    scp = jnp.concatenate(
        [scores[:, _A:, :, :].reshape(B, _AROWS, 128),
         jnp.full((B, _ROWS - _AROWS, 128), -1.0, f32)], axis=1)

    slotg = slot.reshape(_P1_BLKS, 21, 128)
    gx1, gy1, gx2, gy2, gsc = _sc_compact(
        slotg, jnp.asarray(_SRCG_NP), jnp.asarray(_NULL_NP),
        x1d.reshape(_G), y1d.reshape(_G), x2d.reshape(_G), y2d.reshape(_G),
        scp.reshape(_G))

    csc = gsc.reshape(B, _CROWS, 128)
    cx1 = gx1.reshape(B, _CROWS, 128)
    cy1 = gy1.reshape(B, _CROWS, 128)
    cx2 = gx2.reshape(B, _CROWS, 128)
    cy2 = gy2.reshape(B, _CROWS, 128)

    so, bo = pl.pallas_call(
        _stage_b_kernel,
        out_shape=[
            jax.ShapeDtypeStruct((B, 8, 128), f32),
            jax.ShapeDtypeStruct((B, 4, 8, 128), f32),
        ],
    )(csc, cx1, cy1, cx2, cy2)

    s = so.reshape(B, 8 * 128)[:, :_POST_NMS_TOP_N][..., None]
    b = jnp.transpose(bo.reshape(B, 4, 8 * 128)[:, :, :_POST_NMS_TOP_N],
                      (0, 2, 1))
    bcol = jnp.broadcast_to(
        jnp.arange(B, dtype=f32)[:, None, None], (B, _POST_NMS_TOP_N, 1))
    rpn_bbox = jnp.concatenate([bcol, b], axis=2)
    anchors = jnp.asarray(_ANCHORS_NP)
    return s, rpn_bbox, anchors
